# Initial kernel scaffold; baseline (speedup 1.0000x reference)
import jax
import jax.numpy as jnp
from jax.experimental import pallas as pl


def kernel(x, pos, batch, edge_index, Wm1, Wr1, b1, Wm2, Wr2, b2, Wm3, Wr3, b3, Wm4, Wr4, b4, Wm5, Wr5, b5):
    raise NotImplementedError("write your pallas kernel here")


# speedup vs baseline: 1.0055x; 1.0055x over previous
import functools

import jax
import jax.numpy as jnp
from jax.experimental import pallas as pl


def _mm_body(x_ref, w_ref, b_ref, o_ref):
    o_ref[...] = (
        jnp.dot(x_ref[...], w_ref[...], preferred_element_type=jnp.float32)
        + b_ref[...]
    )


def _mm(x, w, b, bn=2048):
    n, k = x.shape
    c = w.shape[1]
    npad = -(-n // bn) * bn
    xp = jnp.pad(x, ((0, npad - n), (0, 0)))
    out = pl.pallas_call(
        _mm_body,
        grid=(npad // bn,),
        in_specs=[
            pl.BlockSpec((bn, k), lambda i: (i, 0)),
            pl.BlockSpec((k, c), lambda i: (0, 0)),
            pl.BlockSpec((1, c), lambda i: (0, 0)),
        ],
        out_specs=pl.BlockSpec((bn, c), lambda i: (i, 0)),
        out_shape=jax.ShapeDtypeStruct((npad, c), jnp.float32),
    )(xp, w, b.reshape(1, c))
    return out[:n]


def _layer(xc, pos, src, dst, wm, wr, b, maxv):
    cin = xc.shape[1]
    c = wm.shape[1]
    w = jnp.concatenate([wm[:cin], wr], axis=1)
    bb = jnp.concatenate([jnp.zeros((c,), jnp.float32), b])
    yr = _mm(xc, w, bb)
    y, r = yr[:, :c], yr[:, c:]
    d = pos[dst, :2] - pos[src, :2]
    ea = jnp.clip(d / (2.0 * maxv) + 0.5, 0.0, 1.0)
    m = y[src] + ea @ wm[cin:]
    agg = jax.ops.segment_max(m, dst, num_segments=xc.shape[0])
    agg = jnp.where(jnp.isfinite(agg), agg, 0.0)
    return jax.nn.relu(agg + r)


def _pool(x, pos, batch, src, dst, gx, gy, bsz, aggr):
    ix = jnp.clip((pos[:, 0] * gx).astype(jnp.int32), 0, gx - 1)
    iy = jnp.clip((pos[:, 1] * gy).astype(jnp.int32), 0, gy - 1)
    cid = batch.astype(jnp.int32) * (gx * gy) + iy * gx + ix
    m = bsz * gx * gy
    ones = jnp.ones((x.shape[0], 1), x.dtype)
    cnt = jax.ops.segment_sum(ones, cid, num_segments=m)
    if aggr == 'max':
        xp = jax.ops.segment_max(x, cid, num_segments=m)
        xp = jnp.where(jnp.isfinite(xp), xp, 0.0)
    else:
        xp = jax.ops.segment_sum(x, cid, num_segments=m) / jnp.maximum(cnt, 1.0)
    posp = jax.ops.segment_sum(pos, cid, num_segments=m) / jnp.maximum(cnt, 1.0)
    batchp = (jnp.arange(m) // (gx * gy)).astype(jnp.int32)
    return xp, posp, batchp, cid[src], cid[dst]


def kernel(x, pos, batch, edge_index, Wm1, Wr1, b1, Wm2, Wr2, b2, Wm3, Wr3,
           b3, Wm4, Wr4, b4, Wm5, Wr5, b5):
    src, dst = edge_index[0], edge_index[1]
    xc = jnp.concatenate([x, pos[:, :2]], axis=1)
    h = _layer(xc, pos, src, dst, Wm1, Wr1, b1, 0.025)
    h, pos, batch, src, dst = _pool(h, pos, batch, src, dst, 56, 40, 4, 'max')
    xc = jnp.concatenate([h, pos[:, :2]], axis=1)
    h = _layer(xc, pos, src, dst, Wm2, Wr2, b2, 0.05)
    h, pos, batch, src, dst = _pool(h, pos, batch, src, dst, 28, 20, 4, 'max')
    xc = jnp.concatenate([h, pos[:, :2]], axis=1)
    h = _layer(xc, pos, src, dst, Wm3, Wr3, b3, 0.1)
    h, pos, batch, src, dst = _pool(h, pos, batch, src, dst, 14, 10, 4, 'max')
    xc = jnp.concatenate([h, pos[:, :2]], axis=1)
    h = _layer(xc, pos, src, dst, Wm4, Wr4, b4, 0.2)
    h, pos, batch, src, dst = _pool(h, pos, batch, src, dst, 7, 5, 4, 'mean')
    xc = jnp.concatenate([h, pos[:, :2]], axis=1)
    h = _layer(xc, pos, src, dst, Wm5, Wr5, b5, 0.4)
    return h
